# baseline (speedup 1.0000x reference)
import functools

import jax
import jax.numpy as jnp
from jax import lax
from jax.experimental import pallas as pl
from jax.experimental.pallas import tpu as pltpu
from jax.experimental.pallas import tpu_sc as plsc

NUM_ROWS = 1000000
NUM_CLASSES = 16
BATCH = 16384

_NC = 2
_NS = 16
_NW = _NC * _NS
_BPW = BATCH // _NW
_G = 16
_NG = _BPW // _G

_mesh = plsc.VectorSubcoreMesh(core_axis_name="c", subcore_axis_name="s")


@functools.partial(
    pl.kernel,
    mesh=_mesh,
    out_type=jax.ShapeDtypeStruct((NUM_CLASSES, BATCH), jnp.float32),
    scratch_types=[
        pltpu.VMEM((_BPW,), jnp.int32),
        pltpu.VMEM((_G, NUM_CLASSES, 128), jnp.float32),
        pltpu.VMEM((NUM_CLASSES, _BPW), jnp.float32),
        pltpu.SemaphoreType.DMA,
    ],
    compiler_params=pltpu.CompilerParams(use_tc_tiling_on_sc=True, needs_layout_passes=False),
)
def _gather_kernel(table_t, idx_hbm, out_t, idx_v, slabs, ostage, sem):
    wid = lax.axis_index("s") * _NC + lax.axis_index("c")
    base = wid * _BPW
    pltpu.sync_copy(idx_hbm.at[pl.ds(base, _BPW)], idx_v)
    c_vec = lax.iota(jnp.int32, _G)

    def body(k, _):
        off = pl.multiple_of(k * _G, _G)
        v = idx_v[pl.ds(off, _G)]
        l_vec = jnp.bitwise_and(v, 127)
        copies = []
        for j in range(_G):
            t0 = pl.multiple_of(jnp.bitwise_and(v[j], -128), 128)
            copies.append(
                pltpu.async_copy(
                    table_t.at[:, pl.ds(t0, 128)], slabs.at[j], sem
                )
            )
        for cp in copies:
            cp.wait()
        for j in range(_G):
            l_s = jnp.broadcast_to(l_vec[j], (_G,))
            vals = plsc.load_gather(slabs.at[j], [c_vec, l_s])
            p_s = jnp.broadcast_to(off + j, (_G,))
            plsc.store_scatter(ostage, [c_vec, p_s], vals)
        return ()

    lax.fori_loop(0, _NG, body, ())
    pltpu.sync_copy(ostage, out_t.at[:, pl.ds(base, _BPW)])


def kernel(probs, x):
    out_t = _gather_kernel(probs.T, x.astype(jnp.int32))
    return out_t.T


# speedup vs baseline: 1.2154x; 1.2154x over previous
import functools

import jax
import jax.numpy as jnp
from jax import lax
from jax.experimental import pallas as pl
from jax.experimental.pallas import tpu as pltpu
from jax.experimental.pallas import tpu_sc as plsc

NUM_ROWS = 1000000
NUM_CLASSES = 16
BATCH = 16384

_NC = 2
_NS = 16
_NW = _NC * _NS
_BPW = BATCH // _NW
_G = 16
_NG = _BPW // _G

_mesh = plsc.VectorSubcoreMesh(core_axis_name="c", subcore_axis_name="s")


@functools.partial(
    pl.kernel,
    mesh=_mesh,
    out_type=jax.ShapeDtypeStruct((NUM_CLASSES, BATCH), jnp.float32),
    scratch_types=[
        pltpu.VMEM((_BPW,), jnp.int32),
        pltpu.VMEM((2, _G, NUM_CLASSES, 128), jnp.float32),
        pltpu.VMEM((NUM_CLASSES, _BPW), jnp.float32),
        pltpu.SemaphoreType.DMA,
        pltpu.SemaphoreType.DMA,
    ],
    compiler_params=pltpu.CompilerParams(use_tc_tiling_on_sc=True, needs_layout_passes=False),
)
def _gather_kernel(table_t, idx_hbm, out_t, idx_v, slabs, ostage, sem_a, sem_b):
    wid = lax.axis_index("s") * _NC + lax.axis_index("c")
    base = wid * _BPW
    pltpu.sync_copy(idx_hbm.at[pl.ds(base, _BPW)], idx_v)
    c_vec = lax.iota(jnp.int32, _G)
    sems = (sem_a, sem_b)

    def fire(k, par):
        off = pl.multiple_of(k * _G, _G)
        v = idx_v[pl.ds(off, _G)]
        for j in range(_G):
            t0 = pl.multiple_of(jnp.bitwise_and(v[j], -128), 128)
            pltpu.async_copy(
                table_t.at[:, pl.ds(t0, 128)], slabs.at[par, j], sems[par]
            )

    def drain_and_extract(k, par):
        for j in range(_G):
            pltpu.make_async_copy(
                table_t.at[:, pl.ds(0, 128)], slabs.at[par, j], sems[par]
            ).wait()
        off = pl.multiple_of(k * _G, _G)
        v = idx_v[pl.ds(off, _G)]
        l_vec = jnp.bitwise_and(v, 127)
        for j in range(_G):
            l_s = jnp.broadcast_to(l_vec[j], (_G,))
            vals = plsc.load_gather(slabs.at[par, j], [c_vec, l_s])
            p_s = jnp.broadcast_to(off + j, (_G,))
            plsc.store_scatter(ostage, [c_vec, p_s], vals)

    fire(0, 0)

    def body(k, _):
        @pl.when(k % 2 == 0)
        def _():
            @pl.when(k + 1 < _NG)
            def _():
                fire(k + 1, 1)

            drain_and_extract(k, 0)

        @pl.when(k % 2 == 1)
        def _():
            @pl.when(k + 1 < _NG)
            def _():
                fire(k + 1, 0)

            drain_and_extract(k, 1)

        return ()

    lax.fori_loop(0, _NG, body, ())
    pltpu.sync_copy(ostage, out_t.at[:, pl.ds(base, _BPW)])


def kernel(probs, x):
    out_t = _gather_kernel(probs.T, x.astype(jnp.int32))
    return out_t.T


# speedup vs baseline: 1.3269x; 1.0917x over previous
import functools

import jax
import jax.numpy as jnp
from jax import lax
from jax.experimental import pallas as pl
from jax.experimental.pallas import tpu as pltpu
from jax.experimental.pallas import tpu_sc as plsc

NUM_ROWS = 1000000
NUM_CLASSES = 16
BATCH = 16384

_NC = 2
_NS = 16
_NW = _NC * _NS
_BPW = BATCH // _NW
_G = 16
_NG = _BPW // _G

_mesh = plsc.VectorSubcoreMesh(core_axis_name="c", subcore_axis_name="s")


@functools.partial(
    pl.kernel,
    mesh=_mesh,
    out_type=jax.ShapeDtypeStruct((NUM_CLASSES, BATCH), jnp.float32),
    scratch_types=[
        pltpu.VMEM((_BPW,), jnp.int32),
        pltpu.VMEM((3, _G, NUM_CLASSES, 128), jnp.float32),
        pltpu.VMEM((NUM_CLASSES, _BPW), jnp.float32),
        pltpu.SemaphoreType.DMA,
        pltpu.SemaphoreType.DMA,
        pltpu.SemaphoreType.DMA,
    ],
    compiler_params=pltpu.CompilerParams(use_tc_tiling_on_sc=True, needs_layout_passes=False),
)
def _gather_kernel(table_t, idx_hbm, out_t, idx_v, slabs, ostage, sem_a, sem_b, sem_c):
    wid = lax.axis_index("s") * _NC + lax.axis_index("c")
    base = wid * _BPW
    pltpu.sync_copy(idx_hbm.at[pl.ds(base, _BPW)], idx_v)
    c_vec = lax.iota(jnp.int32, _G)
    sems = (sem_a, sem_b, sem_c)

    def fire(k, par):
        off = pl.multiple_of(k * _G, _G)
        v = idx_v[pl.ds(off, _G)]
        for j in range(_G):
            t0 = pl.multiple_of(jnp.bitwise_and(v[j], -128), 128)
            pltpu.async_copy(
                table_t.at[:, pl.ds(t0, 128)], slabs.at[par, j], sems[par]
            )

    def drain_and_extract(k, par):
        for j in range(_G):
            pltpu.make_async_copy(
                table_t.at[:, pl.ds(0, 128)], slabs.at[par, j], sems[par]
            ).wait()
        off = pl.multiple_of(k * _G, _G)
        v = idx_v[pl.ds(off, _G)]
        l_vec = jnp.bitwise_and(v, 127)
        for j in range(_G):
            l_s = jnp.broadcast_to(l_vec[j], (_G,))
            vals = plsc.load_gather(slabs.at[par, j], [c_vec, l_s])
            p_s = jnp.broadcast_to(off + j, (_G,))
            plsc.store_scatter(ostage, [c_vec, p_s], vals)

    fire(0, 0)
    fire(1, 1)

    def body(k, _):
        for p in range(3):
            @pl.when(k % 3 == p)
            def _(p=p):
                @pl.when(k + 2 < _NG)
                def _():
                    fire(k + 2, (p + 2) % 3)

                drain_and_extract(k, p)

        return ()

    lax.fori_loop(0, _NG, body, ())
    pltpu.sync_copy(ostage, out_t.at[:, pl.ds(base, _BPW)])


def kernel(probs, x):
    out_t = _gather_kernel(probs.T, x.astype(jnp.int32))
    return out_t.T


# speedup vs baseline: 1.4116x; 1.0638x over previous
import functools

import jax
import jax.numpy as jnp
from jax import lax
from jax.experimental import pallas as pl
from jax.experimental.pallas import tpu as pltpu
from jax.experimental.pallas import tpu_sc as plsc

NUM_ROWS = 1000000
NUM_CLASSES = 16
BATCH = 16384

_NC = 2
_NS = 16
_NW = _NC * _NS
_BPW = BATCH // _NW
_G = 8
_NPAR = 6
_NG = _BPW // _G

_mesh = plsc.VectorSubcoreMesh(core_axis_name="c", subcore_axis_name="s")


@functools.partial(
    pl.kernel,
    mesh=_mesh,
    out_type=jax.ShapeDtypeStruct((NUM_CLASSES, BATCH), jnp.float32),
    scratch_types=[
        pltpu.VMEM((_BPW + 16,), jnp.int32),
        pltpu.VMEM((_NPAR, _G, NUM_CLASSES, 128), jnp.float32),
        pltpu.VMEM((NUM_CLASSES, _BPW), jnp.float32),
    ] + [pltpu.SemaphoreType.DMA] * _NPAR,
    compiler_params=pltpu.CompilerParams(use_tc_tiling_on_sc=True, needs_layout_passes=False),
)
def _gather_kernel(table_t, idx_hbm, out_t, idx_v, slabs, ostage, *sems):
    wid = lax.axis_index("s") * _NC + lax.axis_index("c")
    base = wid * _BPW
    pltpu.sync_copy(idx_hbm.at[pl.ds(base, _BPW)], idx_v.at[pl.ds(0, _BPW)])
    c_vec = lax.iota(jnp.int32, 16)

    def fire(k, par):
        off = pl.multiple_of(k * _G, _G)
        v = idx_v[pl.ds(off, 16)]
        for j in range(_G):
            t0 = pl.multiple_of(jnp.bitwise_and(v[j], -128), 128)
            pltpu.async_copy(
                table_t.at[:, pl.ds(t0, 128)], slabs.at[par, j], sems[par]
            )

    def drain_and_extract(k, par):
        for j in range(_G):
            pltpu.make_async_copy(
                table_t.at[:, pl.ds(0, 128)], slabs.at[par, j], sems[par]
            ).wait()
        off = pl.multiple_of(k * _G, _G)
        v = idx_v[pl.ds(off, 16)]
        l_vec = jnp.bitwise_and(v, 127)
        for j in range(_G):
            l_s = jnp.broadcast_to(l_vec[j], (16,))
            vals = plsc.load_gather(slabs.at[par, j], [c_vec, l_s])
            p_s = jnp.broadcast_to(off + j, (16,))
            plsc.store_scatter(ostage, [c_vec, p_s], vals)

    for g in range(_NPAR - 1):
        fire(g, g)

    def body(k, _):
        for p in range(_NPAR):
            @pl.when(k % _NPAR == p)
            def _(p=p):
                @pl.when(k + _NPAR - 1 < _NG)
                def _():
                    fire(k + _NPAR - 1, (p + _NPAR - 1) % _NPAR)

                drain_and_extract(k, p)

        return ()

    lax.fori_loop(0, _NG, body, ())
    pltpu.sync_copy(ostage, out_t.at[:, pl.ds(base, _BPW)])


def kernel(probs, x):
    out_t = _gather_kernel(probs.T, x.astype(jnp.int32))
    return out_t.T
